# baseline (speedup 1.0000x reference)
import functools

import jax
import jax.numpy as jnp
from jax import lax
from jax.experimental import pallas as pl
from jax.experimental.pallas import tpu as pltpu
from jax.experimental.pallas import tpu_sc as plsc

_LANES = 16


@functools.lru_cache(maxsize=None)
def _build(B, S, E, V):
    info = plsc.get_sparse_core_info()
    nw = info.num_cores * info.num_subcores
    assert B % nw == 0, (B, nw)
    assert E == 64 and S % 8 == 0 and S >= _LANES and V % 2 == 0
    rpw = B // nw
    assert rpw >= 6 and rpw % 2 == 0
    s2 = S // 2
    wide = 2 * E
    n_groups = (S + _LANES - 1) // _LANES
    chunks = []
    off = 0
    while off < S:
        sz = min(128, S - off)
        chunks.append((off, sz))
        off += sz

    mesh = plsc.VectorSubcoreMesh(core_axis_name="c", subcore_axis_name="s")

    @functools.partial(
        pl.kernel,
        mesh=mesh,
        out_type=jax.ShapeDtypeStruct((B, s2, wide), jnp.float32),
        scratch_types=[
            pltpu.VMEM((rpw * S,), jnp.int32),
            pltpu.VMEM((rpw * S,), jnp.int32),
            pltpu.VMEM((2, S, wide), jnp.float32),
            pltpu.VMEM((2, s2, wide), jnp.float32),
            pltpu.VMEM((E, S), jnp.float32),
            pltpu.SemaphoreType.DMA,
            pltpu.SemaphoreType.DMA,
            pltpu.SemaphoreType.DMA,
            pltpu.SemaphoreType.DMA,
        ],
        compiler_params=pltpu.CompilerParams(
            use_tc_tiling_on_sc=True, needs_layout_passes=False),
    )
    def k(x_hbm, tok2_hbm, post_hbm, out_hbm, idx_v, idx2_v, g_v, rows_v,
          post_v, sg0, sg1, ss0, ss1):
        wid = lax.axis_index("s") * info.num_cores + lax.axis_index("c")
        base = wid * rpw
        sem_g = (sg0, sg1)
        sem_s = (ss0, ss1)

        pltpu.sync_copy(x_hbm.at[pl.ds(base * S, rpw * S)], idx_v)
        pltpu.sync_copy(post_hbm, post_v)

        def shift_body(gi, _):
            sl = pl.ds(gi * _LANES, _LANES)
            idx2_v[sl] = lax.shift_right_logical(idx_v[sl], 1)
            return 0
        lax.fori_loop(0, rpw * S // _LANES, shift_body, 0)

        iota = lax.iota(jnp.int32, _LANES)
        half_rows = lax.shift_right_logical(iota, 1)
        half_cols = lax.shift_left(
            lax.bitwise_and(iota, 1), 6)

        def fetch(i, u):
            for off, sz in chunks:
                pltpu.async_copy(
                    tok2_hbm.at[idx2_v.at[pl.ds(i * S + off, sz)]],
                    g_v.at[u].at[pl.ds(off, sz)],
                    sem_g[u])

        def wait_g(u):
            pltpu.make_async_copy(
                tok2_hbm.at[pl.ds(0, S)], g_v.at[u], sem_g[u]).wait()

        def store(i, u):
            pltpu.async_copy(rows_v.at[u], out_hbm.at[base + i], sem_s[u])

        def wait_s(u):
            pltpu.make_async_copy(out_hbm.at[0], rows_v.at[u], sem_s[u]).wait()

        def combine(i, u):
            def body(g, _):
                s0 = lax.min(g * _LANES, S - _LANES)
                xv = idx_v[pl.ds(i * S + s0, _LANES)]
                hv = lax.shift_left(lax.bitwise_and(xv, 1), 6)
                rowc = s0 + iota
                rowsc = lax.shift_right_logical(s0, 1) + half_rows
                for j in range(E):
                    colc = hv + j
                    v = plsc.load_gather(g_v.at[u], [rowc, colc])
                    pv = post_v[j, pl.ds(s0, _LANES)]
                    plsc.store_scatter(
                        rows_v.at[u], [rowsc, half_cols + j], v + pv)
                return 0
            lax.fori_loop(0, n_groups, body, 0)

        fetch(0, 0)
        wait_g(0)
        fetch(1, 1)
        combine(0, 0)
        store(0, 0)

        wait_g(1)
        fetch(2, 0)
        combine(1, 1)
        store(1, 1)

        def group(g, _):
            for u in (0, 1):
                i = 2 + 2 * g + u
                wait_g(u)
                fetch(i + 1, 1 - u)
                wait_s(u)
                combine(i, u)
                store(i, u)
            return 0

        lax.fori_loop(0, (rpw - 4) // 2, group, 0)

        wait_g(0)
        fetch(rpw - 1, 1)
        wait_s(0)
        combine(rpw - 2, 0)
        store(rpw - 2, 0)

        wait_g(1)
        wait_s(1)
        combine(rpw - 1, 1)
        store(rpw - 1, 1)

        wait_s(0)
        wait_s(1)

    return k


def kernel(x, token_table, pos_table):
    B, S = x.shape
    V, E = token_table.shape
    k = _build(B, S, E, V)
    x1 = x.astype(jnp.int32).reshape(B * S)
    tok2 = token_table.reshape(V // 2, 2 * E)
    post = pos_table.T
    out = k(x1, tok2, post)
    return out.reshape(B, S, E)


# speedup vs baseline: 2.3352x; 2.3352x over previous
import functools

import jax
import jax.numpy as jnp
from jax import lax
from jax.experimental import pallas as pl
from jax.experimental.pallas import tpu as pltpu
from jax.experimental.pallas import tpu_sc as plsc

_LANES = 16


@functools.lru_cache(maxsize=None)
def _build(B, S, E, V):
    info = plsc.get_sparse_core_info()
    nw = info.num_cores * info.num_subcores
    assert B % nw == 0, (B, nw)
    assert E == 64 and S % 8 == 0 and S >= _LANES and V % 2 == 0
    rpw = B // nw
    assert rpw >= 6 and rpw % 2 == 0
    s2 = S // 2
    wide = 2 * E
    e_vecs = E // _LANES
    n_groups = (S + _LANES - 1) // _LANES
    s2_pad = (s2 + 7) // 8 * 8
    chunks = []
    off = 0
    while off < S:
        sz = min(128, S - off)
        chunks.append((off, sz))
        off += sz

    mesh = plsc.VectorSubcoreMesh(core_axis_name="c", subcore_axis_name="s")

    @functools.partial(
        pl.kernel,
        mesh=mesh,
        out_type=jax.ShapeDtypeStruct((B, s2, wide), jnp.float32),
        scratch_types=[
            pltpu.VMEM((rpw * S,), jnp.int32),
            pltpu.VMEM((rpw * S,), jnp.int32),
            pltpu.VMEM((2, S, wide), jnp.float32),
            pltpu.VMEM((2, s2_pad, wide), jnp.float32),
            pltpu.VMEM((S, E), jnp.float32),
            pltpu.SemaphoreType.DMA,
            pltpu.SemaphoreType.DMA,
            pltpu.SemaphoreType.DMA,
            pltpu.SemaphoreType.DMA,
        ],
        compiler_params=pltpu.CompilerParams(
            use_tc_tiling_on_sc=True, needs_layout_passes=False),
    )
    def k(x_hbm, tok2_hbm, pos_hbm, out_hbm, idx_v, idx2_v, g_v, rows_v,
          pos_v, sg0, sg1, ss0, ss1):
        wid = lax.axis_index("s") * info.num_cores + lax.axis_index("c")
        base = wid * rpw
        sem_g = (sg0, sg1)
        sem_s = (ss0, ss1)

        pltpu.sync_copy(x_hbm.at[pl.ds(base * S, rpw * S)], idx_v)
        pltpu.sync_copy(pos_hbm, pos_v)

        def shift_body(gi, _):
            sl = pl.ds(gi * _LANES, _LANES)
            idx2_v[sl] = lax.shift_right_logical(idx_v[sl], 1)
            return 0
        lax.fori_loop(0, rpw * S // _LANES, shift_body, 0)

        def fetch(i, u):
            for off, sz in chunks:
                pltpu.async_copy(
                    tok2_hbm.at[idx2_v.at[pl.ds(i * S + off, sz)]],
                    g_v.at[u].at[pl.ds(off, sz)],
                    sem_g[u])

        def wait_g(u):
            pltpu.make_async_copy(
                tok2_hbm.at[pl.ds(0, S)], g_v.at[u], sem_g[u]).wait()

        def store(i, u):
            pltpu.async_copy(
                rows_v.at[u].at[pl.ds(0, s2)], out_hbm.at[base + i], sem_s[u])

        def wait_s(u):
            pltpu.make_async_copy(
                out_hbm.at[0], rows_v.at[u].at[pl.ds(0, s2)], sem_s[u]).wait()

        def combine(i, u):
            def body(g, _):
                s0 = lax.min(g * _LANES, S - _LANES)
                p0 = lax.shift_right_logical(s0, 1)
                xv = idx_v[pl.ds(i * S + s0, _LANES)]
                hv = lax.shift_left(lax.bitwise_and(xv, 1), 6)
                for l in range(_LANES):
                    s = s0 + l
                    p = p0 + l // 2
                    hoff = hv[l]
                    for j in range(e_vecs):
                        v = g_v[u, s, pl.ds(hoff + j * _LANES, _LANES)]
                        pv = pos_v[s, pl.ds(j * _LANES, _LANES)]
                        rows_v[u, p,
                               pl.ds((l % 2) * E + j * _LANES, _LANES)] = v + pv
                return 0
            lax.fori_loop(0, n_groups, body, 0)

        fetch(0, 0)
        wait_g(0)
        fetch(1, 1)
        combine(0, 0)
        store(0, 0)

        wait_g(1)
        fetch(2, 0)
        combine(1, 1)
        store(1, 1)

        def group(g, _):
            for u in (0, 1):
                i = 2 + 2 * g + u
                wait_g(u)
                fetch(i + 1, 1 - u)
                wait_s(u)
                combine(i, u)
                store(i, u)
            return 0

        lax.fori_loop(0, (rpw - 4) // 2, group, 0)

        wait_g(0)
        fetch(rpw - 1, 1)
        wait_s(0)
        combine(rpw - 2, 0)
        store(rpw - 2, 0)

        wait_g(1)
        wait_s(1)
        combine(rpw - 1, 1)
        store(rpw - 1, 1)

        wait_s(0)
        wait_s(1)

    return k


def kernel(x, token_table, pos_table):
    B, S = x.shape
    V, E = token_table.shape
    k = _build(B, S, E, V)
    x1 = x.astype(jnp.int32).reshape(B * S)
    tok2 = token_table.reshape(V // 2, 2 * E)
    out = k(x1, tok2, pos_table)
    return out.reshape(B, S, E)


# speedup vs baseline: 2.6345x; 1.1282x over previous
import functools

import jax
import jax.numpy as jnp
from jax import lax
from jax.experimental import pallas as pl
from jax.experimental.pallas import tpu as pltpu
from jax.experimental.pallas import tpu_sc as plsc

_LANES = 16


@functools.lru_cache(maxsize=None)
def _build(B, S, E, V):
    info = plsc.get_sparse_core_info()
    nw = info.num_cores * info.num_subcores
    assert B % nw == 0, (B, nw)
    assert E % _LANES == 0 and S % 2 == 0
    rpw = B // nw
    assert rpw >= 6 and rpw % 2 == 0
    e_vecs = E // _LANES
    s2 = S // 2
    wide = 2 * E
    chunks = []
    off = 0
    while off < S:
        sz = min(128, S - off)
        chunks.append((off, sz))
        off += sz

    mesh = plsc.VectorSubcoreMesh(core_axis_name="c", subcore_axis_name="s")

    @functools.partial(
        pl.kernel,
        mesh=mesh,
        out_type=jax.ShapeDtypeStruct((B * s2 * wide,), jnp.float32),
        scratch_types=[
            pltpu.VMEM((rpw * S,), jnp.int32),
            pltpu.VMEM((2, S, E), jnp.float32),
            pltpu.VMEM((2, s2 * wide), jnp.float32),
            pltpu.VMEM((s2, wide), jnp.float32),
            pltpu.SemaphoreType.DMA,
            pltpu.SemaphoreType.DMA,
            pltpu.SemaphoreType.DMA,
            pltpu.SemaphoreType.DMA,
        ],
        compiler_params=pltpu.CompilerParams(use_tc_tiling_on_sc=False),
    )
    def k(x_hbm, tok_hbm, pos_hbm, out_hbm, idx_v, g_v, rows_v, pos_v,
          sg0, sg1, ss0, ss1):
        wid = lax.axis_index("s") * info.num_cores + lax.axis_index("c")
        base = wid * rpw
        sem_g = (sg0, sg1)
        sem_s = (ss0, ss1)

        pltpu.sync_copy(x_hbm.at[pl.ds(base * S, rpw * S)], idx_v)
        pltpu.sync_copy(pos_hbm, pos_v)

        def fetch(i, u):
            for off, sz in chunks:
                pltpu.async_copy(
                    tok_hbm.at[idx_v.at[pl.ds(i * S + off, sz)]],
                    g_v.at[u].at[pl.ds(off, sz)],
                    sem_g[u])

        def wait_g(u):
            pltpu.make_async_copy(
                tok_hbm.at[pl.ds(0, S)], g_v.at[u], sem_g[u]).wait()

        blk = s2 * wide

        def store(i, u):
            pltpu.async_copy(
                rows_v.at[u], out_hbm.at[pl.ds((base + i) * blk, blk)],
                sem_s[u])

        def wait_s(u):
            pltpu.make_async_copy(
                out_hbm.at[pl.ds(0, blk)], rows_v.at[u], sem_s[u]).wait()

        def add_pos(u):
            def body(p, _):
                for h in (0, 1):
                    for j in range(e_vecs):
                        src = pl.ds(j * _LANES, _LANES)
                        dst = pl.ds(h * E + j * _LANES, _LANES)
                        fdst = pl.ds(p * wide + h * E + j * _LANES, _LANES)
                        rows_v[u, fdst] = g_v[u, 2 * p + h, src] + pos_v[p, dst]
                return 0
            lax.fori_loop(0, s2, body, 0)

        fetch(0, 0)
        wait_g(0)
        fetch(1, 1)
        add_pos(0)
        store(0, 0)

        wait_g(1)
        fetch(2, 0)
        add_pos(1)
        store(1, 1)

        def group(g, _):
            for u in (0, 1):
                i = 2 + 2 * g + u
                cur = u
                oth = 1 - u
                wait_g(cur)
                fetch(i + 1, oth)
                wait_s(cur)
                add_pos(cur)
                store(i, cur)
            return 0

        lax.fori_loop(0, (rpw - 4) // 2, group, 0)

        wait_g(0)
        fetch(rpw - 1, 1)
        wait_s(0)
        add_pos(0)
        store(rpw - 2, 0)

        wait_g(1)
        wait_s(1)
        add_pos(1)
        store(rpw - 1, 1)

        wait_s(0)
        wait_s(1)

    return k


def kernel(x, token_table, pos_table):
    B, S = x.shape
    V, E = token_table.shape
    k = _build(B, S, E, V)
    pos2 = pos_table.reshape(S // 2, 2 * E)
    x1 = x.astype(jnp.int32).reshape(B * S)
    out = k(x1, token_table, pos2)
    return out.reshape(B, S, E)


# speedup vs baseline: 3.0473x; 1.1567x over previous
import functools

import jax
import jax.numpy as jnp
from jax import lax
from jax.experimental import pallas as pl
from jax.experimental.pallas import tpu as pltpu
from jax.experimental.pallas import tpu_sc as plsc

_LANES = 16


@functools.lru_cache(maxsize=None)
def _build(B, S, E, V):
    info = plsc.get_sparse_core_info()
    nw = info.num_cores * info.num_subcores
    assert B % nw == 0, (B, nw)
    assert E % _LANES == 0 and S % 2 == 0
    rpw = B // nw
    assert rpw >= 6 and rpw % 2 == 0
    e_vecs = E // _LANES
    s2 = S // 2
    wide = 2 * E
    chunks = []
    off = 0
    while off < S:
        sz = min(128, S - off)
        chunks.append((off, sz))
        off += sz

    mesh = plsc.VectorSubcoreMesh(core_axis_name="c", subcore_axis_name="s")

    @functools.partial(
        pl.kernel,
        mesh=mesh,
        out_type=jax.ShapeDtypeStruct((B, s2, wide), jnp.float32),
        scratch_types=[
            pltpu.VMEM((rpw * S,), jnp.int32),
            pltpu.VMEM((2, S, E), jnp.float32),
            pltpu.VMEM((2, s2, wide), jnp.float32),
            pltpu.VMEM((s2, wide), jnp.float32),
            pltpu.SemaphoreType.DMA,
            pltpu.SemaphoreType.DMA,
            pltpu.SemaphoreType.DMA,
            pltpu.SemaphoreType.DMA,
        ],
        compiler_params=pltpu.CompilerParams(use_tc_tiling_on_sc=False),
    )
    def k(x_hbm, tok_hbm, pos_hbm, out_hbm, idx_v, g_v, rows_v, pos_v,
          sg0, sg1, ss0, ss1):
        wid = lax.axis_index("s") * info.num_cores + lax.axis_index("c")
        base = wid * rpw
        sem_g = (sg0, sg1)
        sem_s = (ss0, ss1)

        pltpu.sync_copy(x_hbm.at[pl.ds(base * S, rpw * S)], idx_v)
        pltpu.sync_copy(pos_hbm, pos_v)

        def fetch(i, u):
            for off, sz in chunks:
                pltpu.async_copy(
                    tok_hbm.at[idx_v.at[pl.ds(i * S + off, sz)]],
                    g_v.at[u].at[pl.ds(off, sz)],
                    sem_g[u])

        def wait_g(u):
            pltpu.make_async_copy(
                tok_hbm.at[pl.ds(0, S)], g_v.at[u], sem_g[u]).wait()

        def store(i, u):
            pltpu.async_copy(rows_v.at[u], out_hbm.at[base + i], sem_s[u])

        def wait_s(u):
            pltpu.make_async_copy(out_hbm.at[0], rows_v.at[u], sem_s[u]).wait()

        def add_pos(u):
            def body(p, _):
                for h in (0, 1):
                    for j in range(e_vecs):
                        src = pl.ds(j * _LANES, _LANES)
                        dst = pl.ds(h * E + j * _LANES, _LANES)
                        rows_v[u, p, dst] = g_v[u, 2 * p + h, src] + pos_v[p, dst]
                return 0
            lax.fori_loop(0, s2, body, 0)

        fetch(0, 0)
        wait_g(0)
        fetch(1, 1)
        add_pos(0)
        store(0, 0)

        wait_g(1)
        fetch(2, 0)
        add_pos(1)
        store(1, 1)

        def group(g, _):
            for u in (0, 1):
                i = 2 + 2 * g + u
                cur = u
                oth = 1 - u
                wait_g(cur)
                fetch(i + 1, oth)
                wait_s(cur)
                add_pos(cur)
                store(i, cur)
            return 0

        lax.fori_loop(0, (rpw - 4) // 2, group, 0)

        wait_g(0)
        fetch(rpw - 1, 1)
        wait_s(0)
        add_pos(0)
        store(rpw - 2, 0)

        wait_g(1)
        wait_s(1)
        add_pos(1)
        store(rpw - 1, 1)

        wait_s(0)
        wait_s(1)

    return k


def kernel(x, token_table, pos_table):
    B, S = x.shape
    V, E = token_table.shape
    k = _build(B, S, E, V)
    pos2 = pos_table.reshape(S // 2, 2 * E)
    x1 = x.astype(jnp.int32).reshape(B * S)
    out = k(x1, token_table, pos2)
    return out.reshape(B, S, E)
